# baseline (speedup 1.0000x reference)
import functools

import jax
import jax.numpy as jnp
from jax import lax
from jax.experimental import pallas as pl
from jax.experimental.pallas import tpu as pltpu
from jax.experimental.pallas import tpu_sc as plsc

L_TOKENS = 2048
D_MODEL = 768
N_EXPERTS = 8
EPAD = 128
TILE_M = 128
G_TILES = 40
P_ROWS = G_TILES * TILE_M
NW = 32
ROWS_PER_W = P_ROWS // NW
TOK_PER_W = L_TOKENS // NW
ASG_PER_W = 2 * L_TOKENS // NW
N_SLOTS = 3
_SQRT_HALF = 0.7071067811865476


def _route_meta_kernel(m_ref, h_ref, bias_ref, p_ref, w_ref, cnt_ref):
    logits = lax.dot_general(
        m_ref[...], h_ref[...], (((0,), (1,)), ((), ())),
        preferred_element_type=jnp.float32, precision=lax.Precision.HIGHEST)
    logits = logits + bias_ref[...]
    erow = lax.broadcasted_iota(jnp.int32, (EPAD, L_TOKENS), 0)
    v1 = jnp.max(logits, axis=0, keepdims=True)
    e1 = jnp.min(jnp.where(logits == v1, erow, EPAD), axis=0, keepdims=True)
    l2 = jnp.where(erow == e1, -jnp.inf, logits)
    v2 = jnp.max(l2, axis=0, keepdims=True)
    e2 = jnp.min(jnp.where(l2 == v2, erow, EPAD), axis=0, keepdims=True)
    w1 = 1.0 / (1.0 + jnp.exp(v2 - v1))
    member = jnp.logical_or(erow == e1, erow == e2).astype(jnp.float32)
    t_i = lax.broadcasted_iota(jnp.int32, (L_TOKENS, L_TOKENS), 0)
    t_j = lax.broadcasted_iota(jnp.int32, (L_TOKENS, L_TOKENS), 1)
    ut = (t_i < t_j).astype(jnp.float32)
    rank = lax.dot_general(member, ut, (((1,), (0,)), ((), ())),
                           preferred_element_type=jnp.float32)
    cnt = jnp.sum(member, axis=1, keepdims=True)
    ntiles = jnp.floor((cnt + (TILE_M - 1)) * (1.0 / TILE_M))
    e_i = lax.broadcasted_iota(jnp.int32, (EPAD, EPAD), 0)
    e_j = lax.broadcasted_iota(jnp.int32, (EPAD, EPAD), 1)
    ute = (e_i < e_j).astype(jnp.float32)
    poff = TILE_M * lax.dot_general(ute, ntiles, (((0,), (0,)), ((), ())),
                                    preferred_element_type=jnp.float32)
    base = rank + poff
    p1 = jnp.sum(jnp.where(erow == e1, base, 0.0), axis=0, keepdims=True)
    p2 = jnp.sum(jnp.where(erow == e2, base, 0.0), axis=0, keepdims=True)
    p_ref[0:1, :] = p1.astype(jnp.int32)
    p_ref[1:2, :] = p2.astype(jnp.int32)
    w_ref[0:1, :] = w1
    w_ref[1:2, :] = 1.0 - w1
    cnt_ref[...] = cnt.astype(jnp.int32)


def _sc_dispatch_body(pflat_hbm, h_hbm, x_out, pidx_v, buf, sem):
    wid = lax.axis_index("s") * 2 + lax.axis_index("c")
    abase = wid * ASG_PER_W
    t0 = abase % L_TOKENS
    pltpu.sync_copy(pflat_hbm.at[pl.ds(abase, ASG_PER_W)], pidx_v)
    pltpu.sync_copy(h_hbm.at[pl.ds(t0, ASG_PER_W), :], buf)
    pltpu.async_copy(buf, x_out.at[pidx_v], sem).wait()


def _gmm_kernel(meta_ref, x_ref, b1_ref, b2_ref, w1_hbm, w2_hbm, y_ref,
                w1s, w2s, *sems):
    g = pl.program_id(0)
    te_g = meta_ref[0, g]
    slot = meta_ref[1, g]
    is_new = meta_ref[2, g]
    nx1 = meta_ref[3, g]
    nx2 = meta_ref[4, g]
    sem1 = sems[:N_SLOTS]
    sem2 = sems[N_SLOTS:]

    nch = 4
    c1 = D_MODEL // nch
    c2 = 4 * D_MODEL // nch

    def _start(e, k):
        for c in range(nch):
            pltpu.make_async_copy(w1_hbm.at[e, pl.ds(c * c1, c1), :],
                                  w1s.at[k, pl.ds(c * c1, c1), :],
                                  sem1[k]).start()
            pltpu.make_async_copy(w2_hbm.at[e, pl.ds(c * c2, c2), :],
                                  w2s.at[k, pl.ds(c * c2, c2), :],
                                  sem2[k]).start()

    def _wait(k):
        for c in range(nch):
            pltpu.make_async_copy(w1_hbm.at[0, pl.ds(c * c1, c1), :],
                                  w1s.at[k, pl.ds(c * c1, c1), :],
                                  sem1[k]).wait()
            pltpu.make_async_copy(w2_hbm.at[0, pl.ds(c * c2, c2), :],
                                  w2s.at[k, pl.ds(c * c2, c2), :],
                                  sem2[k]).wait()

    @pl.when(g == 0)
    def _init():
        _start(te_g, 0)

        @pl.when(nx1 >= 0)
        def _():
            _start(nx1, 1)

        @pl.when(nx2 >= 0)
        def _():
            _start(nx2, 2)

    @pl.when(is_new == 1)
    def _on_switch():
        for k in range(N_SLOTS):
            @pl.when(slot == k)
            def _(k=k):
                _wait(k)

        @pl.when(jnp.logical_and(g > 0, nx2 >= 0))
        def _pref():
            tgt = lax.rem(slot + 2, N_SLOTS)
            for k in range(N_SLOTS):
                @pl.when(tgt == k)
                def _(k=k):
                    _start(nx2, k)

    b1v = b1_ref[te_g]
    b2v = b2_ref[te_g]
    for k in range(N_SLOTS):
        @pl.when(slot == k)
        def _compute(k=k):
            hid = jnp.dot(x_ref[...], w1s[k],
                          preferred_element_type=jnp.float32) + b1v
            hid = hid * 0.5 * (1.0 + lax.erf(hid * _SQRT_HALF))
            y_ref[...] = jnp.dot(hid, w2s[k],
                                 preferred_element_type=jnp.float32) + b2v


def _sc_combine_body(p_hbm, w_hbm, y_hbm, out_hbm,
                     p0_v, p1_v, w0_v, w1_v, buf0, buf1, sem0, sem1):
    wid = lax.axis_index("s") * 2 + lax.axis_index("c")
    tbase = wid * TOK_PER_W
    pltpu.sync_copy(p_hbm.at[0, pl.ds(tbase, TOK_PER_W)], p0_v)
    pltpu.sync_copy(p_hbm.at[1, pl.ds(tbase, TOK_PER_W)], p1_v)
    pltpu.sync_copy(w_hbm.at[0, pl.ds(tbase, TOK_PER_W)], w0_v)
    pltpu.sync_copy(w_hbm.at[1, pl.ds(tbase, TOK_PER_W)], w1_v)
    half = TOK_PER_W // 2
    for c in range(2):
        g0 = pltpu.async_copy(y_hbm.at[p0_v.at[pl.ds(c * half, half)]],
                              buf0, sem0)
        g1 = pltpu.async_copy(y_hbm.at[p1_v.at[pl.ds(c * half, half)]],
                              buf1, sem1)
        g0.wait()
        g1.wait()
        unroll = 4

        def addbody(step, carry, c=c):
            for k in range(unroll):
                r = step * unroll + k
                rsp = jnp.zeros((16,), jnp.int32) + (c * half + r)
                w0b = plsc.load_gather(w0_v, [rsp])
                w1b = plsc.load_gather(w1_v, [rsp])
                for cc in range(D_MODEL // 16):
                    sl = pl.ds(cc * 16, 16)
                    buf0[r, sl] = w0b * buf0[r, sl] + w1b * buf1[r, sl]
            return carry

        lax.fori_loop(0, half // unroll, addbody, 0)
        pltpu.sync_copy(buf0, out_hbm.at[pl.ds(tbase + c * half, half), :])


@functools.lru_cache(maxsize=1)
def _sc_kernels():
    mesh = plsc.VectorSubcoreMesh(core_axis_name="c", subcore_axis_name="s")
    dispatch = functools.partial(
        pl.kernel, mesh=mesh,
        out_type=jax.ShapeDtypeStruct((P_ROWS, D_MODEL), jnp.float32),
        scratch_types=[pltpu.VMEM((ASG_PER_W,), jnp.int32),
                       pltpu.VMEM((ASG_PER_W, D_MODEL), jnp.float32),
                       pltpu.SemaphoreType.DMA],
        compiler_params=pltpu.CompilerParams(needs_layout_passes=False),
    )(_sc_dispatch_body)
    combine = functools.partial(
        pl.kernel, mesh=mesh,
        out_type=jax.ShapeDtypeStruct((L_TOKENS, D_MODEL), jnp.float32),
        scratch_types=[pltpu.VMEM((TOK_PER_W,), jnp.int32),
                       pltpu.VMEM((TOK_PER_W,), jnp.int32),
                       pltpu.VMEM((TOK_PER_W,), jnp.float32),
                       pltpu.VMEM((TOK_PER_W,), jnp.float32),
                       pltpu.VMEM((TOK_PER_W // 2, D_MODEL), jnp.float32),
                       pltpu.VMEM((TOK_PER_W // 2, D_MODEL), jnp.float32),
                       pltpu.SemaphoreType.DMA,
                       pltpu.SemaphoreType.DMA],
        compiler_params=pltpu.CompilerParams(needs_layout_passes=False),
    )(_sc_combine_body)
    return dispatch, combine


def _route_meta(h, expert_keys, Wr_v, br_v):
    m = 2.0 * expert_keys.T + Wr_v
    bias = br_v - jnp.sum(expert_keys * expert_keys, axis=1)
    m_pad = jnp.zeros((D_MODEL, EPAD), jnp.float32).at[:, :N_EXPERTS].set(m)
    bias_pad = jnp.full((EPAD, 1), -jnp.inf,
                        jnp.float32).at[:N_EXPERTS, 0].set(bias)
    return pl.pallas_call(
        _route_meta_kernel,
        out_shape=[jax.ShapeDtypeStruct((2, L_TOKENS), jnp.int32),
                   jax.ShapeDtypeStruct((2, L_TOKENS), jnp.float32),
                   jax.ShapeDtypeStruct((EPAD, 1), jnp.int32)],
    )(m_pad, h, bias_pad)


def _tile_expert_map(cnt):
    cnt8 = cnt[:N_EXPERTS, 0]
    nt = (cnt8 + TILE_M - 1) // TILE_M
    cum = jnp.cumsum(nt)
    gidx = jnp.arange(G_TILES, dtype=jnp.int32)
    te = jnp.sum((cum[None, :] <= gidx[:, None]).astype(jnp.int32), axis=1)
    return jnp.minimum(te, N_EXPERTS - 1).astype(jnp.int32)


def _gmm_meta(te):
    g1 = jnp.ones((1,), jnp.int32)
    new = jnp.concatenate([g1, (te[1:] != te[:-1]).astype(jnp.int32)])
    ordi = jnp.cumsum(new) - 1
    slot = ordi % N_SLOTS
    eoo = jnp.full((G_TILES + 2,), -1, jnp.int32).at[ordi].set(te)
    nx1 = eoo[jnp.minimum(ordi + 1, G_TILES + 1)]
    nx2 = eoo[jnp.minimum(ordi + 2, G_TILES + 1)]
    return jnp.stack([te, slot, new, nx1, nx2]).astype(jnp.int32)


def _gmm(meta, x_sorted, W1, b1, W2, b2):
    E = N_EXPERTS
    D = D_MODEL
    grid_spec = pltpu.PrefetchScalarGridSpec(
        num_scalar_prefetch=1,
        grid=(G_TILES,),
        in_specs=[
            pl.BlockSpec((TILE_M, D), lambda g, m: (g, 0)),
            pl.BlockSpec((E, 1, 4 * D), lambda g, m: (0, 0, 0)),
            pl.BlockSpec((E, 1, D), lambda g, m: (0, 0, 0)),
            pl.BlockSpec(memory_space=pl.ANY),
            pl.BlockSpec(memory_space=pl.ANY),
        ],
        out_specs=pl.BlockSpec((TILE_M, D), lambda g, m: (g, 0)),
        scratch_shapes=(
            [pltpu.VMEM((N_SLOTS, D, 4 * D), jnp.float32),
             pltpu.VMEM((N_SLOTS, 4 * D, D), jnp.float32)]
            + [pltpu.SemaphoreType.DMA] * (2 * N_SLOTS)
        ),
    )
    return pl.pallas_call(
        _gmm_kernel,
        grid_spec=grid_spec,
        out_shape=jax.ShapeDtypeStruct((P_ROWS, D), jnp.float32),
        compiler_params=pltpu.CompilerParams(
            dimension_semantics=("arbitrary",),
        ),
    )(meta, x_sorted, b1.reshape(E, 1, 4 * D), b2.reshape(E, 1, D), W1, W2)


def _moe_one_view(h, expert_keys, W1, b1, W2, b2, Wr_v, br_v):
    asg_p, asg_w, cnt = _route_meta(h, expert_keys, Wr_v, br_v)
    te = _tile_expert_map(cnt)
    sc_dispatch, sc_combine = _sc_kernels()
    x_sorted = sc_dispatch(asg_p.reshape(2 * L_TOKENS), h)
    y = _gmm(_gmm_meta(te), x_sorted, W1, b1, W2, b2)
    return sc_combine(asg_p, asg_w, y)


def kernel(views, expert_keys, W1, b1, W2, b2, Wr, br):
    n_views, B, L, D = views.shape
    fused = jnp.zeros((B, L, D), views.dtype)
    for v in range(n_views):
        for b in range(B):
            out = _moe_one_view(views[v, b], expert_keys, W1, b1, W2, b2,
                                Wr[v], br[v])
            fused = fused.at[b].add(out)
    return fused


# speedup vs baseline: 1.0119x; 1.0119x over previous
import functools

import jax
import jax.numpy as jnp
from jax import lax
from jax.experimental import pallas as pl
from jax.experimental.pallas import tpu as pltpu
from jax.experimental.pallas import tpu_sc as plsc

L_TOKENS = 2048
D_MODEL = 768
N_EXPERTS = 8
EPAD = 128
TILE_M = 128
G_TILES = 40
P_ROWS = G_TILES * TILE_M
NW = 32
ROWS_PER_W = P_ROWS // NW
TOK_PER_W = L_TOKENS // NW
ASG_PER_W = 2 * L_TOKENS // NW
N_SLOTS = 3
_SQRT_HALF = 0.7071067811865476


def _route_meta_kernel(k_ref, wr_ref, br_ref, h_ref, p_ref, w_ref, te_ref):
    E = N_EXPERTS
    lg_k = lax.dot_general(k_ref[...], h_ref[...], (((1,), (1,)), ((), ())),
                           preferred_element_type=jnp.float32,
                           precision=lax.Precision.HIGHEST)
    lg_r = lax.dot_general(wr_ref[...], h_ref[...], (((0,), (1,)), ((), ())),
                           preferred_element_type=jnp.float32,
                           precision=lax.Precision.HIGHEST)
    bias = br_ref[...] - jnp.sum(k_ref[...] * k_ref[...], axis=1,
                                 keepdims=True)
    logits = 2.0 * lg_k + lg_r + bias
    erow = lax.broadcasted_iota(jnp.int32, (E, L_TOKENS), 0)
    v1 = jnp.max(logits, axis=0, keepdims=True)
    e1 = jnp.min(jnp.where(logits == v1, erow, E), axis=0, keepdims=True)
    l2 = jnp.where(erow == e1, -jnp.inf, logits)
    v2 = jnp.max(l2, axis=0, keepdims=True)
    e2 = jnp.min(jnp.where(l2 == v2, erow, E), axis=0, keepdims=True)
    w1 = 1.0 / (1.0 + jnp.exp(v2 - v1))
    member = jnp.logical_or(erow == e1, erow == e2).astype(jnp.float32)
    t_i = lax.broadcasted_iota(jnp.int32, (L_TOKENS, L_TOKENS), 0)
    t_j = lax.broadcasted_iota(jnp.int32, (L_TOKENS, L_TOKENS), 1)
    ut = (t_i < t_j).astype(jnp.float32)
    rank = lax.dot_general(member, ut, (((1,), (0,)), ((), ())),
                           preferred_element_type=jnp.float32)
    cnt = jnp.sum(member, axis=1, keepdims=True)
    ntiles = jnp.floor((cnt + (TILE_M - 1)) * (1.0 / TILE_M))
    e_i = lax.broadcasted_iota(jnp.int32, (E, E), 0)
    e_j = lax.broadcasted_iota(jnp.int32, (E, E), 1)
    ute = (e_i < e_j).astype(jnp.float32)
    poff = TILE_M * lax.dot_general(ute, ntiles, (((0,), (0,)), ((), ())),
                                    preferred_element_type=jnp.float32)
    base = rank + poff
    p1 = jnp.sum(jnp.where(erow == e1, base, 0.0), axis=0, keepdims=True)
    p2 = jnp.sum(jnp.where(erow == e2, base, 0.0), axis=0, keepdims=True)
    p_ref[0:1, :] = p1.astype(jnp.int32)
    p_ref[1:2, :] = p2.astype(jnp.int32)
    w_ref[0:1, :] = w1
    w_ref[1:2, :] = 1.0 - w1
    cum_incl = poff * (1.0 / TILE_M) + ntiles
    gcol = lax.broadcasted_iota(jnp.int32, (E, 64), 1).astype(jnp.float32)
    tmap = jnp.sum((cum_incl <= gcol).astype(jnp.int32), axis=0, keepdims=True)
    te_ref[...] = jnp.minimum(tmap, E - 1)


def _sc_dispatch_body(pflat_hbm, h_hbm, x_out, pidx_v, buf, sem):
    wid = lax.axis_index("s") * 2 + lax.axis_index("c")
    abase = wid * ASG_PER_W
    t0 = abase % L_TOKENS
    pltpu.sync_copy(pflat_hbm.at[pl.ds(abase, ASG_PER_W)], pidx_v)
    pltpu.sync_copy(h_hbm.at[pl.ds(t0, ASG_PER_W), :], buf)
    pltpu.async_copy(buf, x_out.at[pidx_v], sem).wait()


def _gmm_kernel(meta_ref, x_ref, b1_ref, b2_ref, w1_hbm, w2_hbm, y_ref,
                w1s, w2s, *sems):
    g = pl.program_id(0)
    te_g = meta_ref[0, g]
    slot = meta_ref[1, g]
    is_new = meta_ref[2, g]
    nx1 = meta_ref[3, g]
    nx2 = meta_ref[4, g]
    sem1 = sems[:N_SLOTS]
    sem2 = sems[N_SLOTS:]

    def _start(e, k):
        pltpu.make_async_copy(w1_hbm.at[e], w1s.at[k], sem1[k]).start()
        pltpu.make_async_copy(w2_hbm.at[e], w2s.at[k], sem2[k]).start()

    def _wait(k):
        pltpu.make_async_copy(w1_hbm.at[0], w1s.at[k], sem1[k]).wait()
        pltpu.make_async_copy(w2_hbm.at[0], w2s.at[k], sem2[k]).wait()

    @pl.when(g == 0)
    def _init():
        _start(te_g, 0)

        @pl.when(nx1 >= 0)
        def _():
            _start(nx1, 1)

        @pl.when(nx2 >= 0)
        def _():
            _start(nx2, 2)

    @pl.when(is_new == 1)
    def _on_switch():
        for k in range(N_SLOTS):
            @pl.when(slot == k)
            def _(k=k):
                _wait(k)

        @pl.when(jnp.logical_and(g > 0, nx2 >= 0))
        def _pref():
            tgt = lax.rem(slot + 2, N_SLOTS)
            for k in range(N_SLOTS):
                @pl.when(tgt == k)
                def _(k=k):
                    _start(nx2, k)

    b1v = b1_ref[te_g]
    b2v = b2_ref[te_g]
    for k in range(N_SLOTS):
        @pl.when(slot == k)
        def _compute(k=k):
            hid = jnp.dot(x_ref[...], w1s[k],
                          preferred_element_type=jnp.float32) + b1v
            hid = hid * 0.5 * (1.0 + lax.erf(hid * _SQRT_HALF))
            y_ref[...] = jnp.dot(hid, w2s[k],
                                 preferred_element_type=jnp.float32) + b2v


def _sc_combine_body(p_hbm, w_hbm, y_hbm, out_hbm,
                     p0_v, p1_v, w0_v, w1_v, b00, b01, b10, b11,
                     s00, s01, s10, s11):
    wid = lax.axis_index("s") * 2 + lax.axis_index("c")
    tbase = wid * TOK_PER_W
    pltpu.sync_copy(p_hbm.at[0, pl.ds(tbase, TOK_PER_W)], p0_v)
    pltpu.sync_copy(p_hbm.at[1, pl.ds(tbase, TOK_PER_W)], p1_v)
    pltpu.sync_copy(w_hbm.at[0, pl.ds(tbase, TOK_PER_W)], w0_v)
    pltpu.sync_copy(w_hbm.at[1, pl.ds(tbase, TOK_PER_W)], w1_v)
    half = TOK_PER_W // 2
    bufs = ((b00, b01, s00, s01), (b10, b11, s10, s11))
    pend = []
    for c in range(2):
        buf0, buf1, sm0, sm1 = bufs[c]
        pend.append((
            pltpu.async_copy(y_hbm.at[p0_v.at[pl.ds(c * half, half)]],
                             buf0, sm0),
            pltpu.async_copy(y_hbm.at[p1_v.at[pl.ds(c * half, half)]],
                             buf1, sm1)))
    for c in range(2):
        buf0, buf1, _, _ = bufs[c]
        g0, g1 = pend[c]
        g0.wait()
        g1.wait()
        unroll = 4

        def addbody(step, carry, c=c, buf0=buf0, buf1=buf1):
            for k in range(unroll):
                r = step * unroll + k
                rsp = jnp.zeros((16,), jnp.int32) + (c * half + r)
                w0b = plsc.load_gather(w0_v, [rsp])
                w1b = plsc.load_gather(w1_v, [rsp])
                for cc in range(D_MODEL // 16):
                    sl = pl.ds(cc * 16, 16)
                    buf0[r, sl] = w0b * buf0[r, sl] + w1b * buf1[r, sl]
            return carry

        lax.fori_loop(0, half // unroll, addbody, 0)
        pltpu.sync_copy(buf0, out_hbm.at[pl.ds(tbase + c * half, half), :])


@functools.lru_cache(maxsize=1)
def _sc_kernels():
    mesh = plsc.VectorSubcoreMesh(core_axis_name="c", subcore_axis_name="s")
    dispatch = functools.partial(
        pl.kernel, mesh=mesh,
        out_type=jax.ShapeDtypeStruct((P_ROWS, D_MODEL), jnp.float32),
        scratch_types=[pltpu.VMEM((ASG_PER_W,), jnp.int32),
                       pltpu.VMEM((ASG_PER_W, D_MODEL), jnp.float32),
                       pltpu.SemaphoreType.DMA],
        compiler_params=pltpu.CompilerParams(needs_layout_passes=False),
    )(_sc_dispatch_body)
    combine = functools.partial(
        pl.kernel, mesh=mesh,
        out_type=jax.ShapeDtypeStruct((L_TOKENS, D_MODEL), jnp.float32),
        scratch_types=[pltpu.VMEM((TOK_PER_W,), jnp.int32),
                       pltpu.VMEM((TOK_PER_W,), jnp.int32),
                       pltpu.VMEM((TOK_PER_W,), jnp.float32),
                       pltpu.VMEM((TOK_PER_W,), jnp.float32),
                       pltpu.VMEM((TOK_PER_W // 2, D_MODEL), jnp.float32),
                       pltpu.VMEM((TOK_PER_W // 2, D_MODEL), jnp.float32),
                       pltpu.VMEM((TOK_PER_W // 2, D_MODEL), jnp.float32),
                       pltpu.VMEM((TOK_PER_W // 2, D_MODEL), jnp.float32),
                       pltpu.SemaphoreType.DMA,
                       pltpu.SemaphoreType.DMA,
                       pltpu.SemaphoreType.DMA,
                       pltpu.SemaphoreType.DMA],
        compiler_params=pltpu.CompilerParams(needs_layout_passes=False),
    )(_sc_combine_body)
    return dispatch, combine


def _route_meta(h, expert_keys, Wr_v, br_v):
    return pl.pallas_call(
        _route_meta_kernel,
        out_shape=[jax.ShapeDtypeStruct((2, L_TOKENS), jnp.int32),
                   jax.ShapeDtypeStruct((2, L_TOKENS), jnp.float32),
                   jax.ShapeDtypeStruct((1, 64), jnp.int32)],
    )(expert_keys, Wr_v, br_v.reshape(N_EXPERTS, 1), h)


def _gmm_meta(te):
    g1 = jnp.ones((1,), jnp.int32)
    new = jnp.concatenate([g1, (te[1:] != te[:-1]).astype(jnp.int32)])
    ordi = jnp.cumsum(new) - 1
    slot = ordi % N_SLOTS
    eoo = jnp.full((G_TILES + 2,), -1, jnp.int32).at[ordi].set(te)
    nx1 = eoo[jnp.minimum(ordi + 1, G_TILES + 1)]
    nx2 = eoo[jnp.minimum(ordi + 2, G_TILES + 1)]
    return jnp.stack([te, slot, new, nx1, nx2]).astype(jnp.int32)


def _gmm(meta, x_sorted, W1, b1, W2, b2):
    E = N_EXPERTS
    D = D_MODEL
    grid_spec = pltpu.PrefetchScalarGridSpec(
        num_scalar_prefetch=1,
        grid=(G_TILES,),
        in_specs=[
            pl.BlockSpec((TILE_M, D), lambda g, m: (g, 0)),
            pl.BlockSpec((E, 1, 4 * D), lambda g, m: (0, 0, 0)),
            pl.BlockSpec((E, 1, D), lambda g, m: (0, 0, 0)),
            pl.BlockSpec(memory_space=pl.ANY),
            pl.BlockSpec(memory_space=pl.ANY),
        ],
        out_specs=pl.BlockSpec((TILE_M, D), lambda g, m: (g, 0)),
        scratch_shapes=(
            [pltpu.VMEM((N_SLOTS, D, 4 * D), jnp.float32),
             pltpu.VMEM((N_SLOTS, 4 * D, D), jnp.float32)]
            + [pltpu.SemaphoreType.DMA] * (2 * N_SLOTS)
        ),
    )
    return pl.pallas_call(
        _gmm_kernel,
        grid_spec=grid_spec,
        out_shape=jax.ShapeDtypeStruct((P_ROWS, D), jnp.float32),
        compiler_params=pltpu.CompilerParams(
            dimension_semantics=("arbitrary",),
        ),
    )(meta, x_sorted, b1.reshape(E, 1, 4 * D), b2.reshape(E, 1, D), W1, W2)


def _moe_one_view(h, expert_keys, W1, b1, W2, b2, Wr_v, br_v):
    asg_p, asg_w, te64 = _route_meta(h, expert_keys, Wr_v, br_v)
    te = te64[0, :G_TILES]
    sc_dispatch, sc_combine = _sc_kernels()
    x_sorted = sc_dispatch(asg_p.reshape(2 * L_TOKENS), h)
    y = _gmm(_gmm_meta(te), x_sorted, W1, b1, W2, b2)
    return sc_combine(asg_p, asg_w, y)


def kernel(views, expert_keys, W1, b1, W2, b2, Wr, br):
    n_views, B, L, D = views.shape
    fused = jnp.zeros((B, L, D), views.dtype)
    for v in range(n_views):
        for b in range(B):
            out = _moe_one_view(views[v, b], expert_keys, W1, b1, W2, b2,
                                Wr[v], br[v])
            fused = fused.at[b].add(out)
    return fused


# speedup vs baseline: 1.0605x; 1.0481x over previous
import functools

import jax
import jax.numpy as jnp
from jax import lax
from jax.experimental import pallas as pl
from jax.experimental.pallas import tpu as pltpu
from jax.experimental.pallas import tpu_sc as plsc

L_TOKENS = 2048
D_MODEL = 768
N_EXPERTS = 8
EPAD = 128
TILE_M = 128
G_TILES = 40
P_ROWS = G_TILES * TILE_M
NW = 32
ROWS_PER_W = P_ROWS // NW
TOK_PER_W = L_TOKENS // NW
ASG_PER_W = 2 * L_TOKENS // NW
N_SLOTS = 3
_SQRT_HALF = 0.7071067811865476


def _route_meta_kernel(m_ref, bias_ref, h_ref, p_ref, w_ref, te_ref):
    E = N_EXPERTS
    logits = lax.dot_general(m_ref[...], h_ref[...], (((1,), (1,)), ((), ())),
                             preferred_element_type=jnp.float32,
                             precision=lax.Precision.HIGHEST)
    logits = logits + bias_ref[...]
    erow = lax.broadcasted_iota(jnp.int32, (E, L_TOKENS), 0)
    v1 = jnp.max(logits, axis=0, keepdims=True)
    e1 = jnp.min(jnp.where(logits == v1, erow, E), axis=0, keepdims=True)
    l2 = jnp.where(erow == e1, -jnp.inf, logits)
    v2 = jnp.max(l2, axis=0, keepdims=True)
    e2 = jnp.min(jnp.where(l2 == v2, erow, E), axis=0, keepdims=True)
    w1 = 1.0 / (1.0 + jnp.exp(v2 - v1))
    member = jnp.logical_or(erow == e1, erow == e2).astype(jnp.float32)
    t_i = lax.broadcasted_iota(jnp.int32, (L_TOKENS, L_TOKENS), 0)
    t_j = lax.broadcasted_iota(jnp.int32, (L_TOKENS, L_TOKENS), 1)
    ut = (t_i < t_j).astype(jnp.float32)
    rank = lax.dot_general(member, ut, (((1,), (0,)), ((), ())),
                           preferred_element_type=jnp.float32)
    cnt = jnp.sum(member, axis=1, keepdims=True)
    ntiles = jnp.floor((cnt + (TILE_M - 1)) * (1.0 / TILE_M))
    e_i = lax.broadcasted_iota(jnp.int32, (E, E), 0)
    e_j = lax.broadcasted_iota(jnp.int32, (E, E), 1)
    ute = (e_i < e_j).astype(jnp.float32)
    poff = TILE_M * lax.dot_general(ute, ntiles, (((0,), (0,)), ((), ())),
                                    preferred_element_type=jnp.float32)
    base = rank + poff
    p1 = jnp.sum(jnp.where(erow == e1, base, 0.0), axis=0, keepdims=True)
    p2 = jnp.sum(jnp.where(erow == e2, base, 0.0), axis=0, keepdims=True)
    p_ref[0:1, 0:L_TOKENS] = p1.astype(jnp.int32)
    p_ref[0:1, L_TOKENS:] = p2.astype(jnp.int32)
    w_ref[0:1, 0:L_TOKENS] = w1
    w_ref[0:1, L_TOKENS:] = 1.0 - w1
    cum_incl = poff * (1.0 / TILE_M) + ntiles
    gcol = lax.broadcasted_iota(jnp.int32, (E, 64), 1).astype(jnp.float32)
    tmap = jnp.sum((cum_incl <= gcol).astype(jnp.int32), axis=0, keepdims=True)
    te_ref[...] = jnp.minimum(tmap, E - 1)


def _sc_dispatch_body(pflat_hbm, h_hbm, x_out, pidx_v, buf, sem):
    wid = lax.axis_index("s") * 2 + lax.axis_index("c")
    abase = wid * ASG_PER_W
    t0 = abase % L_TOKENS
    pltpu.sync_copy(pflat_hbm.at[0, pl.ds(abase, ASG_PER_W)], pidx_v)
    pltpu.sync_copy(h_hbm.at[pl.ds(t0, ASG_PER_W), :], buf)
    pltpu.async_copy(buf, x_out.at[pidx_v], sem).wait()


def _gmm_kernel(meta_ref, x_ref, b1_ref, b2_ref, w1_hbm, w2_hbm, y_ref,
                w1s, w2s, *sems):
    g = pl.program_id(0)
    te_g = meta_ref[0, g]
    slot = meta_ref[1, g]
    is_new = meta_ref[2, g]
    nx1 = meta_ref[3, g]
    nx2 = meta_ref[4, g]
    sem1 = sems[:N_SLOTS]
    sem2 = sems[N_SLOTS:]

    def _start(e, k):
        pltpu.make_async_copy(w1_hbm.at[e], w1s.at[k], sem1[k]).start()
        pltpu.make_async_copy(w2_hbm.at[e], w2s.at[k], sem2[k]).start()

    def _wait(k):
        pltpu.make_async_copy(w1_hbm.at[0], w1s.at[k], sem1[k]).wait()
        pltpu.make_async_copy(w2_hbm.at[0], w2s.at[k], sem2[k]).wait()

    @pl.when(g == 0)
    def _init():
        _start(te_g, 0)

        @pl.when(nx1 >= 0)
        def _():
            _start(nx1, 1)

        @pl.when(nx2 >= 0)
        def _():
            _start(nx2, 2)

    @pl.when(is_new == 1)
    def _on_switch():
        for k in range(N_SLOTS):
            @pl.when(slot == k)
            def _(k=k):
                _wait(k)

        @pl.when(jnp.logical_and(g > 0, nx2 >= 0))
        def _pref():
            tgt = lax.rem(slot + 2, N_SLOTS)
            for k in range(N_SLOTS):
                @pl.when(tgt == k)
                def _(k=k):
                    _start(nx2, k)

    b1v = b1_ref[te_g]
    b2v = b2_ref[te_g]
    for k in range(N_SLOTS):
        @pl.when(slot == k)
        def _compute(k=k):
            hid = jnp.dot(x_ref[...], w1s[k],
                          preferred_element_type=jnp.float32) + b1v
            hid = hid * 0.5 * (1.0 + lax.erf(hid * _SQRT_HALF))
            y_ref[...] = jnp.dot(hid, w2s[k],
                                 preferred_element_type=jnp.float32) + b2v


def _sc_combine_body(p_hbm, w_hbm, y_hbm, out_hbm,
                     p0_v, p1_v, w0_v, w1_v, b00, b01, b10, b11,
                     s00, s01, s10, s11):
    wid = lax.axis_index("s") * 2 + lax.axis_index("c")
    tbase = wid * TOK_PER_W
    pltpu.sync_copy(p_hbm.at[0, pl.ds(tbase, TOK_PER_W)], p0_v)
    pltpu.sync_copy(p_hbm.at[0, pl.ds(L_TOKENS + tbase, TOK_PER_W)], p1_v)
    pltpu.sync_copy(w_hbm.at[0, pl.ds(tbase, TOK_PER_W)], w0_v)
    pltpu.sync_copy(w_hbm.at[0, pl.ds(L_TOKENS + tbase, TOK_PER_W)], w1_v)
    half = TOK_PER_W // 2
    bufs = ((b00, b01, s00, s01), (b10, b11, s10, s11))
    pend = []
    for c in range(2):
        buf0, buf1, sm0, sm1 = bufs[c]
        pend.append((
            pltpu.async_copy(y_hbm.at[p0_v.at[pl.ds(c * half, half)]],
                             buf0, sm0),
            pltpu.async_copy(y_hbm.at[p1_v.at[pl.ds(c * half, half)]],
                             buf1, sm1)))
    wr_pend = []
    for c in range(2):
        buf0, buf1, sm0, _ = bufs[c]
        g0, g1 = pend[c]
        g0.wait()
        g1.wait()
        unroll = 8

        def addbody(step, carry, c=c, buf0=buf0, buf1=buf1):
            for k in range(unroll):
                r = step * unroll + k
                rsp = jnp.zeros((16,), jnp.int32) + (c * half + r)
                w0b = plsc.load_gather(w0_v, [rsp])
                w1b = plsc.load_gather(w1_v, [rsp])
                for cc in range(D_MODEL // 16):
                    sl = pl.ds(cc * 16, 16)
                    buf0[r, sl] = w0b * buf0[r, sl] + w1b * buf1[r, sl]
            return carry

        lax.fori_loop(0, half // unroll, addbody, 0)
        wr_pend.append(pltpu.async_copy(
            buf0, out_hbm.at[pl.ds(tbase + c * half, half), :], sm0))
    for wcp in wr_pend:
        wcp.wait()


@functools.lru_cache(maxsize=1)
def _sc_kernels():
    mesh = plsc.VectorSubcoreMesh(core_axis_name="c", subcore_axis_name="s")
    dispatch = functools.partial(
        pl.kernel, mesh=mesh,
        out_type=jax.ShapeDtypeStruct((P_ROWS, D_MODEL), jnp.float32),
        scratch_types=[pltpu.VMEM((ASG_PER_W,), jnp.int32),
                       pltpu.VMEM((ASG_PER_W, D_MODEL), jnp.float32),
                       pltpu.SemaphoreType.DMA],
        compiler_params=pltpu.CompilerParams(needs_layout_passes=False),
    )(_sc_dispatch_body)
    combine = functools.partial(
        pl.kernel, mesh=mesh,
        out_type=jax.ShapeDtypeStruct((L_TOKENS, D_MODEL), jnp.float32),
        scratch_types=[pltpu.VMEM((TOK_PER_W,), jnp.int32),
                       pltpu.VMEM((TOK_PER_W,), jnp.int32),
                       pltpu.VMEM((TOK_PER_W,), jnp.float32),
                       pltpu.VMEM((TOK_PER_W,), jnp.float32),
                       pltpu.VMEM((TOK_PER_W // 2, D_MODEL), jnp.float32),
                       pltpu.VMEM((TOK_PER_W // 2, D_MODEL), jnp.float32),
                       pltpu.VMEM((TOK_PER_W // 2, D_MODEL), jnp.float32),
                       pltpu.VMEM((TOK_PER_W // 2, D_MODEL), jnp.float32),
                       pltpu.SemaphoreType.DMA,
                       pltpu.SemaphoreType.DMA,
                       pltpu.SemaphoreType.DMA,
                       pltpu.SemaphoreType.DMA],
        compiler_params=pltpu.CompilerParams(needs_layout_passes=False),
    )(_sc_combine_body)
    return dispatch, combine


def _route_meta(h, expert_keys, Wr_v, br_v):
    m = 2.0 * expert_keys + Wr_v.T
    bias = (br_v - jnp.sum(expert_keys * expert_keys, axis=1)).reshape(
        N_EXPERTS, 1)
    return pl.pallas_call(
        _route_meta_kernel,
        out_shape=[jax.ShapeDtypeStruct((1, 2 * L_TOKENS), jnp.int32),
                   jax.ShapeDtypeStruct((1, 2 * L_TOKENS), jnp.float32),
                   jax.ShapeDtypeStruct((1, 64), jnp.int32)],
    )(m, bias, h)


def _gmm_meta(te):
    g1 = jnp.ones((1,), jnp.int32)
    new = jnp.concatenate([g1, (te[1:] != te[:-1]).astype(jnp.int32)])
    ordi = jnp.cumsum(new) - 1
    slot = ordi % N_SLOTS
    eoo = jnp.full((G_TILES + 2,), -1, jnp.int32).at[ordi].set(te)
    nx1 = eoo[jnp.minimum(ordi + 1, G_TILES + 1)]
    nx2 = eoo[jnp.minimum(ordi + 2, G_TILES + 1)]
    return jnp.stack([te, slot, new, nx1, nx2]).astype(jnp.int32)


def _gmm(meta, x_sorted, W1, b1, W2, b2):
    E = N_EXPERTS
    D = D_MODEL
    grid_spec = pltpu.PrefetchScalarGridSpec(
        num_scalar_prefetch=1,
        grid=(G_TILES,),
        in_specs=[
            pl.BlockSpec((TILE_M, D), lambda g, m: (g, 0)),
            pl.BlockSpec((E, 1, 4 * D), lambda g, m: (0, 0, 0)),
            pl.BlockSpec((E, 1, D), lambda g, m: (0, 0, 0)),
            pl.BlockSpec(memory_space=pl.ANY),
            pl.BlockSpec(memory_space=pl.ANY),
        ],
        out_specs=pl.BlockSpec((TILE_M, D), lambda g, m: (g, 0)),
        scratch_shapes=(
            [pltpu.VMEM((N_SLOTS, D, 4 * D), jnp.float32),
             pltpu.VMEM((N_SLOTS, 4 * D, D), jnp.float32)]
            + [pltpu.SemaphoreType.DMA] * (2 * N_SLOTS)
        ),
    )
    return pl.pallas_call(
        _gmm_kernel,
        grid_spec=grid_spec,
        out_shape=jax.ShapeDtypeStruct((P_ROWS, D), jnp.float32),
        compiler_params=pltpu.CompilerParams(
            dimension_semantics=("arbitrary",),
        ),
    )(meta, x_sorted, b1.reshape(E, 1, 4 * D), b2.reshape(E, 1, D), W1, W2)


def _moe_one_view(h, expert_keys, W1, b1, W2, b2, Wr_v, br_v):
    asg_p, asg_w, te64 = _route_meta(h, expert_keys, Wr_v, br_v)
    te = te64[0, :G_TILES]
    sc_dispatch, sc_combine = _sc_kernels()
    x_sorted = sc_dispatch(asg_p, h)
    y = _gmm(_gmm_meta(te), x_sorted, W1, b1, W2, b2)
    return sc_combine(asg_p, asg_w, y)


def kernel(views, expert_keys, W1, b1, W2, b2, Wr, br):
    n_views, B, L, D = views.shape
    fused = jnp.zeros((B, L, D), views.dtype)
    for v in range(n_views):
        for b in range(B):
            out = _moe_one_view(views[v, b], expert_keys, W1, b1, W2, b2,
                                Wr[v], br[v])
            fused = fused.at[b].add(out)
    return fused
